# baseline (speedup 1.0000x reference)
import functools

import jax
import jax.numpy as jnp
from jax import lax
from jax.experimental import pallas as pl
from jax.experimental.pallas import tpu as pltpu
from jax.experimental.pallas import tpu_sc as plsc


def _make_pool(B, L, E, NC, NS, LANES, first, last):
    NW = NC * NS
    assert B % NW == 0 and E % LANES == 0
    b_per_w = B // NW
    NBUF = 4
    assert b_per_w % NBUF == 0
    n_quads = b_per_w // NBUF
    n_acc = E // LANES
    chunks = []
    off = 0
    while off < L:
        n = min(128, L - off)
        chunks.append((off, n))
        off += n

    mesh = plsc.VectorSubcoreMesh(core_axis_name="c", subcore_axis_name="s")

    @functools.partial(
        pl.kernel,
        out_type=jax.ShapeDtypeStruct((B, E), jnp.float32),
        mesh=mesh,
        compiler_params=pltpu.CompilerParams(use_tc_tiling_on_sc=False),
        scratch_types=[
            pltpu.VMEM((b_per_w, L), jnp.int32),
            pltpu.VMEM((L, E), jnp.float32),
            pltpu.VMEM((L, E), jnp.float32),
            pltpu.VMEM((L, E), jnp.float32),
            pltpu.VMEM((L, E), jnp.float32),
            pltpu.VMEM((b_per_w, E), jnp.float32),
            pltpu.SemaphoreType.DMA,
            pltpu.SemaphoreType.DMA,
            pltpu.SemaphoreType.DMA,
            pltpu.SemaphoreType.DMA,
        ],
    )
    def pool(ids_hbm, emb_hbm, *rest):
        if first:
            (out_hbm, idx_v, buf0, buf1, buf2, buf3, out_v,
             s0, s1, s2, s3) = rest
        else:
            (part_hbm, out_hbm, idx_v, buf0, buf1, buf2, buf3, out_v,
             s0, s1, s2, s3) = rest
        wid = lax.axis_index("s") * NC + lax.axis_index("c")
        base = wid * b_per_w
        bufs = (buf0, buf1, buf2, buf3)
        sems = (s0, s1, s2, s3)

        pltpu.sync_copy(ids_hbm.at[pl.ds(base, b_per_w)], idx_v)
        if not first:
            pltpu.sync_copy(part_hbm.at[pl.ds(base, b_per_w)], out_v)

        def descs(b, k):
            return [
                pltpu.make_async_copy(
                    emb_hbm.at[idx_v.at[b, pl.ds(off, n)]],
                    bufs[k].at[pl.ds(off, n)],
                    sems[k],
                )
                for off, n in chunks
            ]

        def issue(b, k):
            for cp in descs(b, k):
                cp.start()

        def drain(b, k):
            for cp in descs(b, k):
                cp.wait()

        def accumulate(b, k):
            buf = bufs[k]

            def acc_body(j, accs):
                return tuple(
                    accs[c] + buf[j, pl.ds(c * LANES, LANES)]
                    for c in range(n_acc)
                )

            if first:
                accs = tuple(
                    jnp.zeros((LANES,), jnp.float32) for _ in range(n_acc)
                )
            else:
                accs = tuple(
                    out_v[b, pl.ds(c * LANES, LANES)] for c in range(n_acc)
                )
            accs = lax.fori_loop(0, L, acc_body, accs, unroll=25)
            scale = jnp.float32(1.0 / L) if last else jnp.float32(1.0)
            for c in range(n_acc):
                out_v[b, pl.ds(c * LANES, LANES)] = accs[c] * scale

        for k in range(NBUF - 1):
            issue(k, k)

        def body(g, carry):
            b0 = NBUF * g
            for k in range(NBUF):
                b = b0 + k
                drain(b, k)

                @pl.when(b + NBUF - 1 < b_per_w)
                def _():
                    issue(b + NBUF - 1, (k + NBUF - 1) % NBUF)

                accumulate(b, k)
            return carry

        lax.fori_loop(0, n_quads, body, 0)
        pltpu.sync_copy(out_v, out_hbm.at[pl.ds(base, b_per_w)])

    return pool


def _mlp(pooled, W1, b1, W2, b2):
    B, E = pooled.shape
    H = W1.shape[0]
    O = W2.shape[0]
    OP = 128
    W2p = jnp.zeros((OP, H), W2.dtype).at[:O].set(W2)
    b2p = jnp.zeros((1, OP), b2.dtype).at[0, :O].set(b2)
    b1r = b1.reshape(1, H)
    BLK = 1024

    def body(x_ref, w1_ref, b1_ref, w2_ref, b2_ref, o_ref):
        x = x_ref[...]
        h = lax.dot_general(
            x, w1_ref[...], (((1,), (1,)), ((), ())),
            preferred_element_type=jnp.float32,
        ) + b1_ref[...]
        h = jnp.maximum(h, 0.0)
        o_ref[...] = lax.dot_general(
            h, w2_ref[...], (((1,), (1,)), ((), ())),
            preferred_element_type=jnp.float32,
        ) + b2_ref[...]

    out = pl.pallas_call(
        body,
        out_shape=jax.ShapeDtypeStruct((B, OP), jnp.float32),
        grid=(B // BLK,),
        in_specs=[
            pl.BlockSpec((BLK, E), lambda i: (i, 0)),
            pl.BlockSpec((H, E), lambda i: (0, 0)),
            pl.BlockSpec((1, H), lambda i: (0, 0)),
            pl.BlockSpec((OP, H), lambda i: (0, 0)),
            pl.BlockSpec((1, OP), lambda i: (0, 0)),
        ],
        out_specs=pl.BlockSpec((BLK, OP), lambda i: (i, 0)),
    )(pooled, W1, b1r, W2p, b2p)
    return out[:, :O]


def kernel(input_ids, emb, W1, b1, W2, b2):
    B, L = input_ids.shape
    V, E = emb.shape
    info = plsc.get_sparse_core_info()
    NC, NS, LANES = info.num_cores, info.num_subcores, info.num_lanes
    H = V // 2
    lo = jnp.pad(emb[:H], ((0, 0), (0, 128 - E))).reshape(V, E)
    hi = jnp.pad(emb[H:], ((0, 0), (0, 128 - E))).reshape(V, E)
    ids = input_ids.astype(jnp.int32)
    ids_lo = jnp.where(ids < H, ids * 2, 1)
    ids_hi = jnp.where(ids >= H, (ids - H) * 2, 1)
    part = _make_pool(B, L, E, NC, NS, LANES, True, False)(ids_lo, lo)
    pooled = _make_pool(B, L, E, NC, NS, LANES, False, True)(
        ids_hi, hi, part)
    return _mlp(pooled, W1, b1, W2, b2)


# speedup vs baseline: 26.0974x; 26.0974x over previous
import functools

import jax
import jax.numpy as jnp
from jax import lax
from jax.experimental import pallas as pl
from jax.experimental.pallas import tpu as pltpu
from jax.experimental.pallas import tpu_sc as plsc


def _make_pool(B, L, E, NC, NS, LANES):
    NW = NC * NS
    assert B % NW == 0 and E % LANES == 0
    b_per_w = B // NW
    NBUF = 4
    assert b_per_w % NBUF == 0
    n_quads = b_per_w // NBUF
    n_acc = E // LANES
    chunks = []
    off = 0
    while off < L:
        n = min(128, L - off)
        chunks.append((off, n))
        off += n

    mesh = plsc.VectorSubcoreMesh(core_axis_name="c", subcore_axis_name="s")

    @functools.partial(
        pl.kernel,
        out_type=jax.ShapeDtypeStruct((B, E), jnp.float32),
        mesh=mesh,
        compiler_params=pltpu.CompilerParams(use_tc_tiling_on_sc=False),
        scratch_types=[
            pltpu.VMEM((b_per_w, L), jnp.int32),
            pltpu.VMEM((L, E), jnp.float32),
            pltpu.VMEM((L, E), jnp.float32),
            pltpu.VMEM((L, E), jnp.float32),
            pltpu.VMEM((L, E), jnp.float32),
            pltpu.VMEM((b_per_w, E), jnp.float32),
            pltpu.SemaphoreType.DMA,
            pltpu.SemaphoreType.DMA,
            pltpu.SemaphoreType.DMA,
            pltpu.SemaphoreType.DMA,
        ],
    )
    def pool(ids_hbm, emb_hbm, out_hbm, idx_v,
             buf0, buf1, buf2, buf3, out_v, s0, s1, s2, s3):
        wid = lax.axis_index("s") * NC + lax.axis_index("c")
        base = wid * b_per_w
        bufs = (buf0, buf1, buf2, buf3)
        sems = (s0, s1, s2, s3)

        pltpu.sync_copy(ids_hbm.at[pl.ds(base, b_per_w)], idx_v)

        def descs(b, k):
            return [
                pltpu.make_async_copy(
                    emb_hbm.at[idx_v.at[b, pl.ds(off, n)]],
                    bufs[k].at[pl.ds(off, n)],
                    sems[k],
                )
                for off, n in chunks
            ]

        def issue(b, k):
            for cp in descs(b, k):
                cp.start()

        def drain(b, k):
            for cp in descs(b, k):
                cp.wait()

        def accumulate(b, k):
            buf = bufs[k]

            def acc_body(j, accs):
                return tuple(
                    accs[c] + buf[j, pl.ds(c * LANES, LANES)]
                    for c in range(n_acc)
                )

            accs = tuple(
                jnp.zeros((LANES,), jnp.float32) for _ in range(n_acc)
            )
            accs = lax.fori_loop(0, L, acc_body, accs, unroll=25)
            scale = jnp.float32(1.0 / L)
            for c in range(n_acc):
                out_v[b, pl.ds(c * LANES, LANES)] = accs[c] * scale

        for k in range(NBUF - 1):
            issue(k, k)

        def body(g, carry):
            b0 = NBUF * g
            for k in range(NBUF):
                b = b0 + k
                drain(b, k)

                @pl.when(b + NBUF - 1 < b_per_w)
                def _():
                    issue(b + NBUF - 1, (k + NBUF - 1) % NBUF)

                accumulate(b, k)
            return carry

        lax.fori_loop(0, n_quads, body, 0)
        pltpu.sync_copy(out_v, out_hbm.at[pl.ds(base, b_per_w)])

    return pool


def _mlp(pooled, W1, b1, W2, b2):
    B, E = pooled.shape
    H = W1.shape[0]
    O = W2.shape[0]
    OP = 128
    W2p = jnp.zeros((OP, H), W2.dtype).at[:O].set(W2)
    b2p = jnp.zeros((1, OP), b2.dtype).at[0, :O].set(b2)
    b1r = b1.reshape(1, H)
    BLK = 1024

    def body(x_ref, w1_ref, b1_ref, w2_ref, b2_ref, o_ref):
        x = x_ref[...]
        h = lax.dot_general(
            x, w1_ref[...], (((1,), (1,)), ((), ())),
            preferred_element_type=jnp.float32,
        ) + b1_ref[...]
        h = jnp.maximum(h, 0.0)
        o_ref[...] = lax.dot_general(
            h, w2_ref[...], (((1,), (1,)), ((), ())),
            preferred_element_type=jnp.float32,
        ) + b2_ref[...]

    out = pl.pallas_call(
        body,
        out_shape=jax.ShapeDtypeStruct((B, OP), jnp.float32),
        grid=(B // BLK,),
        in_specs=[
            pl.BlockSpec((BLK, E), lambda i: (i, 0)),
            pl.BlockSpec((H, E), lambda i: (0, 0)),
            pl.BlockSpec((1, H), lambda i: (0, 0)),
            pl.BlockSpec((OP, H), lambda i: (0, 0)),
            pl.BlockSpec((1, OP), lambda i: (0, 0)),
        ],
        out_specs=pl.BlockSpec((BLK, OP), lambda i: (i, 0)),
    )(pooled, W1, b1r, W2p, b2p)
    return out[:, :O]


def kernel(input_ids, emb, W1, b1, W2, b2):
    B, L = input_ids.shape
    V, E = emb.shape
    info = plsc.get_sparse_core_info()
    NC, NS, LANES = info.num_cores, info.num_subcores, info.num_lanes
    embp = jnp.pad(emb, ((0, 0), (0, 128 - E)))
    emb2 = embp.reshape(2 * V, E)
    pool = _make_pool(B, L, E, NC, NS, LANES)
    pooled = pool(input_ids.astype(jnp.int32) * 2, emb2)
    return _mlp(pooled, W1, b1, W2, b2)
